# baseline (speedup 1.0000x reference)
import functools

import jax
import jax.numpy as jnp
from jax import lax
from jax.experimental import pallas as pl
from jax.experimental.pallas import tpu as pltpu
from jax.experimental.pallas import tpu_sc as plsc

VOCAB = 100000
D = 1024
BATCH = 4
SEQ = 8192
TOT = BATCH * SEQ

_info = plsc.get_sparse_core_info()
NC = _info.num_cores
NS = _info.num_subcores
NW = NC * NS
BPW = TOT // NW
CH = 16
NCHUNK = BPW // CH
NBUF = 4
LOOKAHEAD = 2
NROUND = NCHUNK // NBUF

_mesh = plsc.VectorSubcoreMesh(core_axis_name="c", subcore_axis_name="s")


@functools.partial(
    pl.kernel,
    mesh=_mesh,
    out_type=jax.ShapeDtypeStruct((TOT, D), jnp.float32),
    scratch_types=[
        pltpu.VMEM((NCHUNK, CH), jnp.int32),
        pltpu.VMEM((NBUF, CH, D), jnp.float32),
        pltpu.SemaphoreType.DMA,
        pltpu.SemaphoreType.DMA,
        pltpu.SemaphoreType.DMA,
        pltpu.SemaphoreType.DMA,
        pltpu.SemaphoreType.DMA,
        pltpu.SemaphoreType.DMA,
        pltpu.SemaphoreType.DMA,
        pltpu.SemaphoreType.DMA,
    ],
)
def _embed(idx_hbm, table_hbm, out_hbm, idx_v, bufs,
           g0, g1, g2, g3, s0, s1, s2, s3):
    wid = lax.axis_index("s") * NC + lax.axis_index("c")
    base = wid * BPW
    gsems = (g0, g1, g2, g3)
    ssems = (s0, s1, s2, s3)

    def gather(c, b):
        return pltpu.make_async_copy(table_hbm.at[idx_v.at[c]], bufs.at[b],
                                     gsems[b])

    def store(c, b):
        return pltpu.make_async_copy(
            bufs.at[b], out_hbm.at[pl.ds(base + c * CH, CH)], ssems[b])

    pltpu.sync_copy(idx_hbm.at[wid], idx_v)

    for c in range(LOOKAHEAD):
        gather(c, c % NBUF).start()

    def round_body(o, _):
        for b in range(NBUF):
            c = o * NBUF + b
            cn = c + LOOKAHEAD

            @pl.when(cn < NCHUNK)
            def _issue_next():
                bn = (b + LOOKAHEAD) % NBUF

                @pl.when(cn >= NBUF)
                def _free_buf():
                    store(cn - NBUF, bn).wait()

                gather(cn, bn).start()

            gather(c, b).wait()
            store(c, b).start()
        return _

    lax.fori_loop(0, NROUND, round_body, None)

    for c in range(NCHUNK - NBUF, NCHUNK):
        store(c, c % NBUF).wait()


def kernel(input_ids, table):
    ids = input_ids.reshape(NW, NCHUNK, CH).astype(jnp.int32)
    out = _embed(ids, table)
    return out.reshape(BATCH, SEQ, D)
